# Initial kernel scaffold; baseline (speedup 1.0000x reference)
import functools

import jax
import jax.numpy as jnp
from jax import lax
from jax.experimental import pallas as pl
from jax.experimental.pallas import tpu as pltpu
from jax.experimental.pallas import tpu_sc as plsc

D = 128
K = 512
B = 1024
COMMIT_W = 0.25

NC, NS, L = 2, 16, 16
NW = NC * NS
DPW = D // NW
GROUPS = B // L

_mesh = plsc.VectorSubcoreMesh(core_axis_name="c", subcore_axis_name="s")


@functools.partial(
    pl.kernel,
    out_type=(
        jax.ShapeDtypeStruct((D, B), jnp.float32),
        jax.ShapeDtypeStruct((D, B), jnp.int32),
        jax.ShapeDtypeStruct((NW, L), jnp.float32),
    ),
    mesh=_mesh,
    scratch_types=[
        pltpu.VMEM((DPW, B), jnp.float32),
        pltpu.VMEM((DPW, K), jnp.float32),
        pltpu.VMEM((DPW, K), jnp.int32),
        pltpu.VMEM((DPW, B), jnp.float32),
        pltpu.VMEM((DPW, B), jnp.int32),
        pltpu.VMEM((L,), jnp.float32),
    ],
)
def _vq_search(zt_hbm, ws_hbm, wp_hbm, zq_hbm, idx_hbm, loss_hbm,
               z_v, ws_v, wp_v, zq_v, idx_v, ls_v):
    wid = lax.axis_index("s") * NC + lax.axis_index("c")
    d0 = wid * DPW
    pltpu.sync_copy(zt_hbm.at[pl.ds(d0, DPW)], z_v)
    pltpu.sync_copy(ws_hbm.at[pl.ds(d0, DPW)], ws_v)
    pltpu.sync_copy(wp_hbm.at[pl.ds(d0, DPW)], wp_v)

    acc_total = jnp.zeros((L,), jnp.float32)
    for dloc in range(DPW):
        dsplat = jnp.full((L,), dloc, jnp.int32)

        def body(g, acc, dloc=dloc, dsplat=dsplat):
            zv = z_v[dloc, pl.ds(g * L, L)]
            lo = jnp.full((L,), -1, jnp.int32)
            hi = jnp.full((L,), K, jnp.int32)
            for _ in range(9):
                mid = (lo + hi) >> 1
                w = plsc.load_gather(ws_v, [dsplat, mid])
                cond = w < zv
                lo = jnp.where(cond, mid, lo)
                hi = jnp.where(cond, hi, mid)
            lft = jnp.maximum(hi - 1, 0)
            rgt = jnp.minimum(hi, K - 1)
            wl = plsc.load_gather(ws_v, [dsplat, lft])
            wr = plsc.load_gather(ws_v, [dsplat, rgt])
            il = plsc.load_gather(wp_v, [dsplat, lft])
            ir = plsc.load_gather(wp_v, [dsplat, rgt])
            dl = jnp.abs(zv - wl)
            dr = jnp.abs(zv - wr)
            takel = (dl < dr) | ((dl == dr) & (il <= ir))
            wq = jnp.where(takel, wl, wr)
            iq = jnp.where(takel, il, ir)
            zq_v[dloc, pl.ds(g * L, L)] = wq
            idx_v[dloc, pl.ds(g * L, L)] = iq
            dd = zv - wq
            return acc + dd * dd

        acc_total = lax.fori_loop(0, GROUPS, body, acc_total)

    pltpu.sync_copy(zq_v, zq_hbm.at[pl.ds(d0, DPW)])
    pltpu.sync_copy(idx_v, idx_hbm.at[pl.ds(d0, DPW)])
    ls_v[...] = acc_total
    pltpu.sync_copy(ls_v, loss_hbm.at[wid])


def kernel(z, W):
    Wsq = W[:, :, 0]
    order = jnp.argsort(Wsq, axis=1)
    ws = jnp.take_along_axis(Wsq, order, axis=1)
    wp = order.astype(jnp.int32)
    zt = z.T
    zq_t, idx_t, part = _vq_search(zt, ws, wp)
    z_q = zq_t.T
    indices = idx_t.T
    vq_loss = (1.0 + COMMIT_W) * jnp.sum(part) / B
    z_q_sg = z + (z_q - z)
    return (z_q_sg, vq_loss, indices.astype(jnp.int64))


# speedup vs baseline: 17.4128x; 17.4128x over previous
import functools

import jax
import jax.numpy as jnp
from jax import lax
from jax.experimental import pallas as pl
from jax.experimental.pallas import tpu as pltpu
from jax.experimental.pallas import tpu_sc as plsc

D = 128
K = 512
B = 1024
COMMIT_W = 0.25

NC, NS, L = 2, 16, 16
NW = NC * NS
DPW = D // NW
GROUPS = B // L

_mesh = plsc.VectorSubcoreMesh(core_axis_name="c", subcore_axis_name="s")


@functools.partial(
    pl.kernel,
    out_type=(
        jax.ShapeDtypeStruct((D * B,), jnp.float32),
        jax.ShapeDtypeStruct((D * B,), jnp.int32),
        jax.ShapeDtypeStruct((NW, L), jnp.float32),
    ),
    mesh=_mesh,
    compiler_params=pltpu.CompilerParams(needs_layout_passes=False),
    scratch_types=[
        pltpu.VMEM((DPW * B,), jnp.float32),
        pltpu.VMEM((DPW * K,), jnp.float32),
        pltpu.VMEM((DPW * K,), jnp.int32),
        pltpu.VMEM((DPW * B,), jnp.float32),
        pltpu.VMEM((DPW * B,), jnp.int32),
        pltpu.VMEM((L,), jnp.float32),
    ],
)
def _vq_search(zt_hbm, ws_hbm, wp_hbm, zq_hbm, idx_hbm, loss_hbm,
               z_v, ws_v, wp_v, zq_v, idx_v, ls_v):
    wid = lax.axis_index("s") * NC + lax.axis_index("c")
    d0 = wid * DPW
    pltpu.sync_copy(zt_hbm.at[pl.ds(d0 * B, DPW * B)], z_v)
    pltpu.sync_copy(ws_hbm.at[pl.ds(d0 * K, DPW * K)], ws_v)
    pltpu.sync_copy(wp_hbm.at[pl.ds(d0 * K, DPW * K)], wp_v)

    acc_total = jnp.zeros((L,), jnp.float32)
    for dloc in range(DPW):
        dbase = jnp.full((L,), dloc * K, jnp.int32)

        def body(g, acc, dloc=dloc, dbase=dbase):
            zv = z_v[pl.ds(dloc * B + g * L, L)]
            lo = jnp.full((L,), -1, jnp.int32)
            hi = jnp.full((L,), K, jnp.int32)
            for _ in range(9):
                mid = (lo + hi) >> 1
                w = plsc.load_gather(ws_v, [dbase + mid])
                cond = w < zv
                lo = jnp.where(cond, mid, lo)
                hi = jnp.where(cond, hi, mid)
            best_d = jnp.full((L,), jnp.inf, jnp.float32)
            best_p = jnp.full((L,), 2**31 - 1, jnp.int32)
            best_w = jnp.zeros((L,), jnp.float32)
            for off in range(-3, 3):
                c = jnp.minimum(jnp.maximum(hi + off, 0), K - 1)
                w = plsc.load_gather(ws_v, [dbase + c])
                p = plsc.load_gather(wp_v, [dbase + c])
                dc = zv - w
                dc = dc * dc
                better = (dc < best_d) | ((dc == best_d) & (p < best_p))
                best_d = jnp.where(better, dc, best_d)
                best_p = jnp.where(better, p, best_p)
                best_w = jnp.where(better, w, best_w)
            zq_v[pl.ds(dloc * B + g * L, L)] = best_w
            idx_v[pl.ds(dloc * B + g * L, L)] = best_p
            dd = zv - best_w
            return acc + dd * dd

        acc_total = lax.fori_loop(0, GROUPS, body, acc_total)

    pltpu.sync_copy(zq_v, zq_hbm.at[pl.ds(d0 * B, DPW * B)])
    pltpu.sync_copy(idx_v, idx_hbm.at[pl.ds(d0 * B, DPW * B)])
    ls_v[...] = acc_total
    pltpu.sync_copy(ls_v, loss_hbm.at[wid])


def kernel(z, W):
    Wsq = W[:, :, 0]
    order = jnp.argsort(Wsq, axis=1)
    ws = jnp.take_along_axis(Wsq, order, axis=1)
    wp = order.astype(jnp.int32)
    zt = z.T
    zq_f, idx_f, part = _vq_search(
        zt.reshape(D * B), ws.reshape(D * K), wp.reshape(D * K))
    z_q = zq_f.reshape(D, B).T
    indices = idx_f.reshape(D, B).T
    vq_loss = (1.0 + COMMIT_W) * jnp.sum(part) / B
    z_q_sg = z + (z_q - z)
    return (z_q_sg, vq_loss, indices.astype(jnp.int64))


# speedup vs baseline: 24.7770x; 1.4229x over previous
import functools

import jax
import jax.numpy as jnp
from jax import lax
from jax.experimental import pallas as pl
from jax.experimental.pallas import tpu as pltpu
from jax.experimental.pallas import tpu_sc as plsc

D = 128
K = 512
B = 1024
COMMIT_W = 0.25

NC, NS, L = 2, 16, 16
NW = NC * NS
DPW = D // NW
GROUPS = B // L

_mesh = plsc.VectorSubcoreMesh(core_axis_name="c", subcore_axis_name="s")


@functools.partial(
    pl.kernel,
    out_type=(
        jax.ShapeDtypeStruct((D * B,), jnp.float32),
        jax.ShapeDtypeStruct((D * B,), jnp.int32),
        jax.ShapeDtypeStruct((NW, L), jnp.float32),
    ),
    mesh=_mesh,
    compiler_params=pltpu.CompilerParams(needs_layout_passes=False),
    scratch_types=[
        pltpu.VMEM((DPW * B,), jnp.float32),
        pltpu.VMEM((DPW * K,), jnp.float32),
        pltpu.VMEM((DPW * K,), jnp.int32),
        pltpu.VMEM((DPW * B,), jnp.float32),
        pltpu.VMEM((DPW * B,), jnp.int32),
        pltpu.VMEM((L,), jnp.float32),
    ],
)
def _vq_search(zt_hbm, ws_hbm, wp_hbm, zq_hbm, idx_hbm, loss_hbm,
               z_v, ws_v, wp_v, zq_v, idx_v, ls_v):
    wid = lax.axis_index("s") * NC + lax.axis_index("c")
    d0 = wid * DPW
    pltpu.sync_copy(zt_hbm.at[pl.ds(d0 * B, DPW * B)], z_v)
    pltpu.sync_copy(ws_hbm.at[pl.ds(d0 * K, DPW * K)], ws_v)
    pltpu.sync_copy(wp_hbm.at[pl.ds(d0 * K, DPW * K)], wp_v)

    UNROLL = 4
    acc_total = jnp.zeros((L,), jnp.float32)
    for dloc in range(DPW):
        dbase = jnp.full((L,), dloc * K, jnp.int32)

        def body(t, acc, dloc=dloc, dbase=dbase):
            offs = [dloc * B + (t * UNROLL + u) * L for u in range(UNROLL)]
            zs = [z_v[pl.ds(o, L)] for o in offs]
            los = [jnp.full((L,), -1, jnp.int32) for _ in range(UNROLL)]
            his = [jnp.full((L,), K, jnp.int32) for _ in range(UNROLL)]
            for _ in range(9):
                for u in range(UNROLL):
                    mid = (los[u] + his[u]) >> 1
                    w = plsc.load_gather(ws_v, [dbase + mid])
                    cond = w < zs[u]
                    los[u] = jnp.where(cond, mid, los[u])
                    his[u] = jnp.where(cond, his[u], mid)
            for u in range(UNROLL):
                zv, hi = zs[u], his[u]
                best_d = jnp.full((L,), jnp.inf, jnp.float32)
                best_p = jnp.full((L,), 2**31 - 1, jnp.int32)
                best_w = jnp.zeros((L,), jnp.float32)
                for off in range(-3, 3):
                    c = jnp.minimum(jnp.maximum(hi + off, 0), K - 1)
                    w = plsc.load_gather(ws_v, [dbase + c])
                    p = plsc.load_gather(wp_v, [dbase + c])
                    dc = zv - w
                    dc = dc * dc
                    better = (dc < best_d) | ((dc == best_d) & (p < best_p))
                    best_d = jnp.where(better, dc, best_d)
                    best_p = jnp.where(better, p, best_p)
                    best_w = jnp.where(better, w, best_w)
                zq_v[pl.ds(offs[u], L)] = best_w
                idx_v[pl.ds(offs[u], L)] = best_p
                dd = zv - best_w
                acc = acc + dd * dd
            return acc

        acc_total = lax.fori_loop(0, GROUPS // UNROLL, body, acc_total)

    pltpu.sync_copy(zq_v, zq_hbm.at[pl.ds(d0 * B, DPW * B)])
    pltpu.sync_copy(idx_v, idx_hbm.at[pl.ds(d0 * B, DPW * B)])
    ls_v[...] = acc_total
    pltpu.sync_copy(ls_v, loss_hbm.at[wid])


def kernel(z, W):
    Wsq = W[:, :, 0]
    iota = jax.lax.broadcasted_iota(jnp.int32, (D, K), 1)
    ws, wp = lax.sort((Wsq, iota), dimension=1, is_stable=True, num_keys=1)
    zt = z.T
    zq_f, idx_f, part = _vq_search(
        zt.reshape(D * B), ws.reshape(D * K), wp.reshape(D * K))
    z_q = zq_f.reshape(D, B).T
    indices = idx_f.reshape(D, B).T
    vq_loss = (1.0 + COMMIT_W) * jnp.sum(part) / B
    z_q_sg = z + (z_q - z)
    return (z_q_sg, vq_loss, indices.astype(jnp.int64))


# speedup vs baseline: 25.1729x; 1.0160x over previous
import functools

import jax
import jax.numpy as jnp
from jax import lax
from jax.experimental import pallas as pl
from jax.experimental.pallas import tpu as pltpu
from jax.experimental.pallas import tpu_sc as plsc

D = 128
K = 512
B = 1024
COMMIT_W = 0.25

NC, NS, L = 2, 16, 16
NW = NC * NS
DPW = D // NW
GROUPS = B // L

_mesh = plsc.VectorSubcoreMesh(core_axis_name="c", subcore_axis_name="s")


@functools.partial(
    pl.kernel,
    out_type=(
        jax.ShapeDtypeStruct((D * B,), jnp.float32),
        jax.ShapeDtypeStruct((D * B,), jnp.int32),
        jax.ShapeDtypeStruct((NW, L), jnp.float32),
    ),
    mesh=_mesh,
    compiler_params=pltpu.CompilerParams(needs_layout_passes=False),
    scratch_types=[
        pltpu.VMEM((DPW * B,), jnp.float32),
        pltpu.VMEM((DPW * K,), jnp.float32),
        pltpu.VMEM((DPW * K,), jnp.int32),
        pltpu.VMEM((DPW * B,), jnp.float32),
        pltpu.VMEM((DPW * B,), jnp.int32),
        pltpu.VMEM((L,), jnp.float32),
    ],
)
def _vq_search(zt_hbm, ws_hbm, wp_hbm, zq_hbm, idx_hbm, loss_hbm,
               z_v, ws_v, wp_v, zq_v, idx_v, ls_v):
    wid = lax.axis_index("s") * NC + lax.axis_index("c")
    d0 = wid * DPW
    pltpu.sync_copy(zt_hbm.at[pl.ds(d0 * B, DPW * B)], z_v)
    pltpu.sync_copy(ws_hbm.at[pl.ds(d0 * K, DPW * K)], ws_v)
    pltpu.sync_copy(wp_hbm.at[pl.ds(d0 * K, DPW * K)], wp_v)

    UNROLL = 8
    acc_total = jnp.zeros((L,), jnp.float32)
    for dloc in range(DPW):
        dbase = jnp.full((L,), dloc * K, jnp.int32)

        def body(t, acc, dloc=dloc, dbase=dbase):
            offs = [dloc * B + (t * UNROLL + u) * L for u in range(UNROLL)]
            zs = [z_v[pl.ds(o, L)] for o in offs]
            los = [jnp.full((L,), -1, jnp.int32) for _ in range(UNROLL)]
            his = [jnp.full((L,), K, jnp.int32) for _ in range(UNROLL)]
            for _ in range(9):
                for u in range(UNROLL):
                    mid = (los[u] + his[u]) >> 1
                    w = plsc.load_gather(ws_v, [dbase + mid])
                    cond = w < zs[u]
                    los[u] = jnp.where(cond, mid, los[u])
                    his[u] = jnp.where(cond, his[u], mid)
            for u in range(UNROLL):
                zv, hi = zs[u], his[u]
                best_d = jnp.full((L,), jnp.inf, jnp.float32)
                best_p = jnp.full((L,), 2**31 - 1, jnp.int32)
                best_w = jnp.zeros((L,), jnp.float32)
                for off in range(-3, 3):
                    c = jnp.minimum(jnp.maximum(hi + off, 0), K - 1)
                    w = plsc.load_gather(ws_v, [dbase + c])
                    p = plsc.load_gather(wp_v, [dbase + c])
                    dc = zv - w
                    dc = dc * dc
                    better = (dc < best_d) | ((dc == best_d) & (p < best_p))
                    best_d = jnp.where(better, dc, best_d)
                    best_p = jnp.where(better, p, best_p)
                    best_w = jnp.where(better, w, best_w)
                zq_v[pl.ds(offs[u], L)] = best_w
                idx_v[pl.ds(offs[u], L)] = best_p
                dd = zv - best_w
                acc = acc + dd * dd
            return acc

        acc_total = lax.fori_loop(0, GROUPS // UNROLL, body, acc_total)

    pltpu.sync_copy(zq_v, zq_hbm.at[pl.ds(d0 * B, DPW * B)])
    pltpu.sync_copy(idx_v, idx_hbm.at[pl.ds(d0 * B, DPW * B)])
    ls_v[...] = acc_total
    pltpu.sync_copy(ls_v, loss_hbm.at[wid])


def kernel(z, W):
    Wsq = W[:, :, 0]
    iota = jax.lax.broadcasted_iota(jnp.int32, (D, K), 1)
    ws, wp = lax.sort((Wsq, iota), dimension=1, is_stable=True, num_keys=1)
    zt = z.T
    zq_f, idx_f, part = _vq_search(
        zt.reshape(D * B), ws.reshape(D * K), wp.reshape(D * K))
    z_q = zq_f.reshape(D, B).T
    indices = idx_f.reshape(D, B).T
    vq_loss = (1.0 + COMMIT_W) * jnp.sum(part) / B
    z_q_sg = z + (z_q - z)
    return (z_q_sg, vq_loss, indices.astype(jnp.int64))


# speedup vs baseline: 28.2613x; 1.1227x over previous
import functools

import jax
import jax.numpy as jnp
from jax import lax
from jax.experimental import pallas as pl
from jax.experimental.pallas import tpu as pltpu
from jax.experimental.pallas import tpu_sc as plsc

D = 128
K = 512
B = 1024
COMMIT_W = 0.25

NC, NS, L = 2, 16, 16
NW = NC * NS
DPW = D // NW
GROUPS = B // L

_mesh = plsc.VectorSubcoreMesh(core_axis_name="c", subcore_axis_name="s")


@functools.partial(
    pl.kernel,
    out_type=(
        jax.ShapeDtypeStruct((D * B,), jnp.float32),
        jax.ShapeDtypeStruct((D * B,), jnp.int32),
        jax.ShapeDtypeStruct((NW, L), jnp.float32),
    ),
    mesh=_mesh,
    compiler_params=pltpu.CompilerParams(needs_layout_passes=False),
    scratch_types=[
        pltpu.VMEM((DPW * B,), jnp.float32),
        pltpu.VMEM((DPW * K,), jnp.float32),
        pltpu.VMEM((DPW * K,), jnp.int32),
        pltpu.VMEM((DPW * B,), jnp.float32),
        pltpu.VMEM((DPW * B,), jnp.int32),
        pltpu.VMEM((L,), jnp.float32),
    ],
)
def _vq_search(zt_hbm, ws_hbm, wp_hbm, zq_hbm, idx_hbm, loss_hbm,
               z_v, ws_v, wp_v, zq_v, idx_v, ls_v):
    wid = lax.axis_index("s") * NC + lax.axis_index("c")
    d0 = wid * DPW
    pltpu.sync_copy(zt_hbm.at[pl.ds(d0 * B, DPW * B)], z_v)
    pltpu.sync_copy(ws_hbm.at[pl.ds(d0 * K, DPW * K)], ws_v)
    pltpu.sync_copy(wp_hbm.at[pl.ds(d0 * K, DPW * K)], wp_v)

    UNROLL = 4
    acc_total = jnp.zeros((L,), jnp.float32)
    for dloc in range(DPW):
        dbase = jnp.full((L,), dloc * K, jnp.int32)
        def splat(ref, pos):
            return plsc.load_gather(ref, [jnp.full((L,), pos, jnp.int32)])

        w_lo = [splat(ws_v, dloc * K + i) for i in range(3)]
        p_lo = [splat(wp_v, dloc * K + i) for i in range(3)]
        w_hi = [splat(ws_v, dloc * K + K - 3 + i) for i in range(3)]
        p_hi = [splat(wp_v, dloc * K + K - 3 + i) for i in range(3)]

        def pick(zv, cands):
            best_d = jnp.full((L,), jnp.inf, jnp.float32)
            best_p = jnp.full((L,), 2**31 - 1, jnp.int32)
            best_w = jnp.zeros((L,), jnp.float32)
            for w, p in cands:
                dc = zv - w
                dc = dc * dc
                better = (dc < best_d) | ((dc == best_d) & (p < best_p))
                best_d = jnp.where(better, dc, best_d)
                best_p = jnp.where(better, p, best_p)
                best_w = jnp.where(better, w, best_w)
            return best_w, best_p

        def slow_group(zv, out_hi, dbase=dbase):
            lo = jnp.full((L,), -1, jnp.int32)
            hi = jnp.full((L,), K, jnp.int32)
            for _ in range(9):
                mid = (lo + hi) >> 1
                w = plsc.load_gather(ws_v, [dbase + mid])
                cond = w < zv
                lo = jnp.where(cond, mid, lo)
                hi = jnp.where(cond, hi, mid)
            cands = []
            for off in range(-3, 3):
                c = jnp.minimum(jnp.maximum(hi + off, 0), K - 1)
                cands.append((plsc.load_gather(ws_v, [dbase + c]),
                              plsc.load_gather(wp_v, [dbase + c])))
            return pick(zv, cands)

        def fast_group(zv, out_hi, w_lo=w_lo, p_lo=p_lo, w_hi=w_hi, p_hi=p_hi):
            cands = [(jnp.where(out_hi, w_hi[i], w_lo[2 - i]),
                      jnp.where(out_hi, p_hi[i], p_lo[2 - i]))
                     for i in range(3)]
            return pick(zv, cands)

        def body(t, acc, dloc=dloc, w_lo0=w_lo[0], w_hi2=w_hi[2]):
            for u in range(UNROLL):
                o = dloc * B + (t * UNROLL + u) * L
                zv = z_v[pl.ds(o, L)]
                out_hi = zv > w_hi2
                any_in = jnp.logical_not(jnp.all((zv < w_lo0) | out_hi))
                best_w, best_p = lax.cond(any_in, slow_group, fast_group,
                                          zv, out_hi)
                zq_v[pl.ds(o, L)] = best_w
                idx_v[pl.ds(o, L)] = best_p
                dd = zv - best_w
                acc = acc + dd * dd
            return acc

        acc_total = lax.fori_loop(0, GROUPS // UNROLL, body, acc_total)

    pltpu.sync_copy(zq_v, zq_hbm.at[pl.ds(d0 * B, DPW * B)])
    pltpu.sync_copy(idx_v, idx_hbm.at[pl.ds(d0 * B, DPW * B)])
    ls_v[...] = acc_total
    pltpu.sync_copy(ls_v, loss_hbm.at[wid])


def kernel(z, W):
    Wsq = W[:, :, 0]
    iota = jax.lax.broadcasted_iota(jnp.int32, (D, K), 1)
    ws, wp = lax.sort((Wsq, iota), dimension=1, is_stable=True, num_keys=1)
    zt = z.T
    zq_f, idx_f, part = _vq_search(
        zt.reshape(D * B), ws.reshape(D * K), wp.reshape(D * K))
    z_q = zq_f.reshape(D, B).T
    indices = idx_f.reshape(D, B).T
    vq_loss = (1.0 + COMMIT_W) * jnp.sum(part) / B
    z_q_sg = z + (z_q - z)
    return (z_q_sg, vq_loss, indices.astype(jnp.int64))
